# baseline (speedup 1.0000x reference)
import functools

import jax
import jax.numpy as jnp
from jax import lax
from jax.experimental import pallas as pl
from jax.experimental.pallas import tpu as pltpu
from jax.experimental.pallas import tpu_sc as plsc

N = 10000
D = 128
E = 320000

NC = 2
NS = 16
L = 16
NW = NC * NS

BLK = 128
NB = 80
EPT = NB * BLK
EPAD = EPT * NW
NP = 10240
PHB = 20
RPT = NP // NS

_MESH = plsc.VectorSubcoreMesh(
    core_axis_name="c", subcore_axis_name="s", num_cores=NC, num_subcores=NS)
_SC_PARAMS = pltpu.CompilerParams(needs_layout_passes=False, use_tc_tiling_on_sc=False)


def _zero_rows(rows):
    def body(r, _):
        for j in range(D // L):
            rows[r, pl.ds(j * L, L)] = jnp.zeros((L,), jnp.float32)
        return 0
    lax.fori_loop(0, BLK, body, 0)


def _deg_body(idx_hbm, degp_hbm, idx_all, deg_l):
    c = lax.axis_index("c")
    s = lax.axis_index("s")
    wid = s * NC + c

    pltpu.sync_copy(idx_hbm.at[pl.ds(wid * NB * 3, NB * 3)], idx_all)

    def zero(i, _):
        deg_l[pl.ds(i * L, L)] = jnp.zeros((L,), jnp.float32)
        return 0
    lax.fori_loop(0, NP // L, zero, 0)

    def blk(b, _):
        def vec(i, _):
            idx = idx_all[3 * b + 1, pl.ds(i * L, L)]
            val = plsc.bitcast(idx_all[3 * b + 2, pl.ds(i * L, L)],
                               jnp.float32)
            plsc.addupdate_scatter(deg_l, [idx], val)
            return 0
        lax.fori_loop(0, BLK // L, vec, 0)
        return 0
    lax.fori_loop(0, NB, blk, 0)

    pltpu.sync_copy(deg_l, degp_hbm.at[wid])


_deg_kernel = functools.partial(
    pl.kernel,
    out_type=jax.ShapeDtypeStruct((NW, NP), jnp.float32),
    mesh=_MESH,
    compiler_params=_SC_PARAMS,
    scratch_types=[
        pltpu.VMEM((NB * 3, BLK), jnp.int32),
        pltpu.VMEM((NP,), jnp.float32),
    ],
)(_deg_body)


HB = NB // 2


def _agg_body(idx_hbm, h_hbm, aggp_hbm,
              acc_sh, idx_all, rbf, rf0, rf1, sg, ss0, ss1):
    c = lax.axis_index("c")
    s = lax.axis_index("s")

    _zero_rows(rf0)
    for k in range(RPT // BLK):
        pltpu.sync_copy(rf0, acc_sh.at[pl.ds(s * RPT + k * BLK, BLK)])
    plsc.subcore_barrier()

    def gather(b):
        pltpu.async_copy(h_hbm.at[idx_all.at[3 * b]], rbf, sg)

    def wait_gather(b):
        pltpu.make_async_copy(h_hbm.at[idx_all.at[3 * b]], rbf, sg).wait()

    def scatter(b, rf, sem):
        pltpu.async_copy(rf, acc_sh.at[idx_all.at[3 * b + 1]], sem, add=True)

    def wait_scatter(b, rf, sem):
        pltpu.make_async_copy(rf, acc_sh.at[idx_all.at[3 * b + 1]],
                              sem).wait()

    def scale(b, rf):
        def grp(g, _):
            wv = plsc.bitcast(idx_all[3 * b + 2, pl.ds(g * L, L)],
                              jnp.float32)
            for r in range(L):
                nrm = wv[r]
                e = g * L + r
                for j in range(D // (2 * L)):
                    ab = plsc.bitcast(rbf[e, pl.ds(j * L, L)], jnp.bfloat16)
                    av, bv = plsc.unpack(ab, format=plsc.PackFormat.INTERLEAVED)
                    rf[e, pl.ds(j * 2 * L, L)] = av * nrm
                    rf[e, pl.ds(j * 2 * L + L, L)] = bv * nrm
            return 0
        lax.fori_loop(0, BLK // L, grp, 0)

    wid = s * NC + c
    base_rows = wid * (NB * 3)

    def phase(p):
        pltpu.sync_copy(
            idx_hbm.at[pl.ds(base_rows + p * (PHB * 3), PHB * 3)], idx_all)
        gather(0)

        def body(k, _):
            l0 = 2 * k
            l1 = l0 + 1
            wait_gather(l0)

            @pl.when(k > 0)
            def _():
                wait_scatter(l0 - 2, rf0, ss0)
            scale(l0, rf0)
            gather(l1)
            scatter(l0, rf0, ss0)
            wait_gather(l1)

            @pl.when(k > 0)
            def _():
                wait_scatter(l1 - 2, rf1, ss1)
            scale(l1, rf1)

            @pl.when(k < PHB // 2 - 1)
            def _():
                gather(l0 + 2)
            scatter(l1, rf1, ss1)
            return 0
        lax.fori_loop(0, PHB // 2, body, 0)
        wait_scatter(PHB - 2, rf0, ss0)
        wait_scatter(PHB - 1, rf1, ss1)

    for p in range(NB // PHB):
        phase(p)

    plsc.subcore_barrier()
    pltpu.sync_copy(acc_sh.at[pl.ds(s * RPT, RPT)],
                    aggp_hbm.at[c, pl.ds(s * RPT, RPT)])


_agg_kernel = functools.partial(
    pl.kernel,
    out_type=jax.ShapeDtypeStruct((NC, NP, D), jnp.float32),
    mesh=_MESH,
    compiler_params=_SC_PARAMS,
    scratch_types=[
        pltpu.VMEM_SHARED((NP, D), jnp.float32),
        pltpu.VMEM((PHB * 3, BLK), jnp.int32),
        pltpu.VMEM((BLK, D // 2), jnp.int32),
        pltpu.VMEM((BLK, D), jnp.float32),
        pltpu.VMEM((BLK, D), jnp.float32),
        pltpu.SemaphoreType.DMA,
        pltpu.SemaphoreType.DMA,
        pltpu.SemaphoreType.DMA,
    ],
)(_agg_body)


def _dinv_body(degp_ref, dinv_ref):
    deg = jnp.sum(degp_ref[...], axis=0) + 1.0
    dinv_ref[...] = jnp.where(deg > 0, lax.rsqrt(deg), 0.0)


_dinv = pl.pallas_call(
    _dinv_body,
    out_shape=jax.ShapeDtypeStruct((NP // 128, 128), jnp.float32))


def _mm1_body(x_ref, w_ref, dinv_ref, h_ref):
    h_ref[...] = dinv_ref[...] * jnp.dot(x_ref[...], w_ref[...],
                                         preferred_element_type=jnp.float32)


_mm1 = pl.pallas_call(
    _mm1_body,
    out_shape=jax.ShapeDtypeStruct((N, D), jnp.float32))


def _mid_body(aggp_ref, h_ref, dinv_ref, b_ref, w_ref, h2_ref):
    z = dinv_ref[...] * (aggp_ref[0, :N] + aggp_ref[1, :N] + h_ref[...])
    z = jnp.maximum(z + b_ref[...], 0.0)
    h2_ref[...] = dinv_ref[...] * jnp.dot(z, w_ref[...],
                                          preferred_element_type=jnp.float32)


_mid = pl.pallas_call(
    _mid_body,
    out_shape=jax.ShapeDtypeStruct((N, D), jnp.float32))


def _out_body(aggp_ref, h_ref, dinv_ref, b_ref, x_ref, out_ref):
    z = dinv_ref[...] * (aggp_ref[0, :N] + aggp_ref[1, :N] + h_ref[...])
    out_ref[:, :D] = jnp.maximum(z + b_ref[...], 0.0)
    out_ref[:, D:] = x_ref[...]


_out = pl.pallas_call(
    _out_body,
    out_shape=jax.ShapeDtypeStruct((N, 2 * D), jnp.float32))


def kernel(x, edge_index, edge_weight, W1, b1, W2, b2):
    src = edge_index[0].astype(jnp.int32)
    dst = edge_index[1].astype(jnp.int32)
    w = edge_weight.astype(jnp.float32)
    pad = EPAD - E
    src_p = jnp.pad(src, (0, pad)).reshape(NW * NB, 1, BLK)
    dst_fill = N + (jnp.arange(pad, dtype=jnp.int32) % (NP - N))
    dst_p = jnp.concatenate([dst, dst_fill]).reshape(NW * NB, 1, BLK)
    wbits = lax.bitcast_convert_type(jnp.pad(w, (0, pad)),
                                     jnp.int32).reshape(NW * NB, 1, BLK)
    packed = jnp.concatenate([src_p, dst_p, wbits],
                             axis=1).reshape(NW * NB * 3, BLK)

    degp = _deg_kernel(packed)
    dinv2d = _dinv(degp.reshape(NW, NP // 128, 128))
    dinv_col = dinv2d.reshape(NP, 1)[:N]

    def to_bf16_perm(h):
        hb = (h.astype(jnp.bfloat16)
              .reshape(N, D // 32, 2, 16)
              .transpose(0, 1, 3, 2)
              .reshape(N, D // 2, 2))
        return lax.bitcast_convert_type(hb, jnp.int32)

    h1 = _mm1(x, W1, dinv_col)
    p = _agg_kernel(packed, to_bf16_perm(h1))
    h2 = _mid(p, h1, dinv_col, b1.reshape(1, D), W2)
    q = _agg_kernel(packed, to_bf16_perm(h2))
    return _out(q, h2, dinv_col, b2.reshape(1, D), x)


# speedup vs baseline: 1.5132x; 1.5132x over previous
import functools

import jax
import jax.numpy as jnp
from jax import lax
from jax.experimental import pallas as pl
from jax.experimental.pallas import tpu as pltpu
from jax.experimental.pallas import tpu_sc as plsc

N = 10000
D = 128
E = 320000

NC = 2
NS = 16
L = 16
NW = NC * NS

BLK = 128
NB = 80
EPT = NB * BLK
EPAD = EPT * NW
NP = 10240
PHB = 40
RPT = NP // NS

_MESH = plsc.VectorSubcoreMesh(
    core_axis_name="c", subcore_axis_name="s", num_cores=NC, num_subcores=NS)
_SC_PARAMS = pltpu.CompilerParams(needs_layout_passes=False, use_tc_tiling_on_sc=False)


def _zero_rows(rows):
    def body(r, _):
        for j in range(D // L):
            rows[r, pl.ds(j * L, L)] = jnp.zeros((L,), jnp.float32)
        return 0
    lax.fori_loop(0, BLK, body, 0)


def _deg_body(idx_hbm, degp_hbm, idx_all, deg_l):
    c = lax.axis_index("c")
    s = lax.axis_index("s")
    wid = s * NC + c

    pltpu.sync_copy(idx_hbm.at[pl.ds(wid * NB * 3, NB * 3)], idx_all)

    def zero(i, _):
        deg_l[pl.ds(i * L, L)] = jnp.zeros((L,), jnp.float32)
        return 0
    lax.fori_loop(0, NP // L, zero, 0)

    def blk(b, _):
        def vec(i, _):
            idx = idx_all[3 * b + 1, pl.ds(i * L, L)]
            val = plsc.bitcast(idx_all[3 * b + 2, pl.ds(i * L, L)],
                               jnp.float32)
            plsc.addupdate_scatter(deg_l, [idx], val)
            return 0
        lax.fori_loop(0, BLK // L, vec, 0)
        return 0
    lax.fori_loop(0, NB, blk, 0)

    pltpu.sync_copy(deg_l, degp_hbm.at[wid])


_deg_kernel = functools.partial(
    pl.kernel,
    out_type=jax.ShapeDtypeStruct((NW, NP), jnp.float32),
    mesh=_MESH,
    compiler_params=_SC_PARAMS,
    scratch_types=[
        pltpu.VMEM((NB * 3, BLK), jnp.int32),
        pltpu.VMEM((NP,), jnp.float32),
    ],
)(_deg_body)


HB = NB // 2


def _agg_body(idx_hbm, h_hbm, aggp_hbm,
              acc_sh, idx_all, rbf0, rbf1, rf, sg0, sg1, ss):
    c = lax.axis_index("c")
    s = lax.axis_index("s")

    _zero_rows(rf)
    for k in range(RPT // BLK):
        pltpu.sync_copy(rf, acc_sh.at[pl.ds(s * RPT + k * BLK, BLK)])
    plsc.subcore_barrier()

    def gather(b, rows, sem):
        pltpu.async_copy(h_hbm.at[idx_all.at[3 * b]], rows, sem)

    def wait_gather(b, rows, sem):
        pltpu.make_async_copy(h_hbm.at[idx_all.at[3 * b]], rows, sem).wait()

    def scatter(b):
        pltpu.async_copy(rf, acc_sh.at[idx_all.at[3 * b + 1]], ss, add=True)

    def wait_scatter(b):
        pltpu.make_async_copy(rf, acc_sh.at[idx_all.at[3 * b + 1]],
                              ss).wait()

    def scale(b, rbf):
        def grp(g, _):
            wv = plsc.bitcast(idx_all[3 * b + 2, pl.ds(g * L, L)],
                              jnp.float32)
            for r in range(L):
                nrm = wv[r]
                e = g * L + r
                for j in range(D // (2 * L)):
                    ab = plsc.bitcast(rbf[e, pl.ds(j * L, L)], jnp.bfloat16)
                    av, bv = plsc.unpack(ab, format=plsc.PackFormat.INTERLEAVED)
                    rf[e, pl.ds(j * 2 * L, L)] = av * nrm
                    rf[e, pl.ds(j * 2 * L + L, L)] = bv * nrm
            return 0
        lax.fori_loop(0, BLK // L, grp, 0)

    wid = s * NC + c
    base_rows = wid * (NB * 3)

    def phase(p, first):
        pltpu.sync_copy(
            idx_hbm.at[pl.ds(base_rows + p * (PHB * 3), PHB * 3)], idx_all)
        gather(0, rbf0, sg0)
        gather(1, rbf1, sg1)

        def body(k, _):
            l0 = 2 * k
            l1 = l0 + 1
            wait_gather(l0, rbf0, sg0)

            @pl.when(jnp.logical_or(k > 0, jnp.logical_not(first)))
            def _():
                wait_scatter(l1 - 2)
            scale(l0, rbf0)
            scatter(l0)

            @pl.when(k < PHB // 2 - 1)
            def _():
                gather(l0 + 2, rbf0, sg0)
            wait_gather(l1, rbf1, sg1)
            wait_scatter(l0)
            scale(l1, rbf1)
            scatter(l1)

            @pl.when(k < PHB // 2 - 1)
            def _():
                gather(l1 + 2, rbf1, sg1)
            return 0
        lax.fori_loop(0, PHB // 2, body, 0)

    phase(0, jnp.bool_(True))
    for p in range(1, NB // PHB):
        phase(p, jnp.bool_(False))
    wait_scatter(PHB - 1)

    plsc.subcore_barrier()
    pltpu.sync_copy(acc_sh.at[pl.ds(s * RPT, RPT)],
                    aggp_hbm.at[c, pl.ds(s * RPT, RPT)])


_agg_kernel = functools.partial(
    pl.kernel,
    out_type=jax.ShapeDtypeStruct((NC, NP, D), jnp.float32),
    mesh=_MESH,
    compiler_params=_SC_PARAMS,
    scratch_types=[
        pltpu.VMEM_SHARED((NP, D), jnp.float32),
        pltpu.VMEM((PHB * 3, BLK), jnp.int32),
        pltpu.VMEM((BLK, D // 2), jnp.int32),
        pltpu.VMEM((BLK, D // 2), jnp.int32),
        pltpu.VMEM((BLK, D), jnp.float32),
        pltpu.SemaphoreType.DMA,
        pltpu.SemaphoreType.DMA,
        pltpu.SemaphoreType.DMA,
    ],
)(_agg_body)


def _dinv_body(degp_ref, dinv_ref):
    deg = jnp.sum(degp_ref[...], axis=0) + 1.0
    dinv_ref[...] = jnp.where(deg > 0, lax.rsqrt(deg), 0.0)


_dinv = pl.pallas_call(
    _dinv_body,
    out_shape=jax.ShapeDtypeStruct((NP // 128, 128), jnp.float32))


def _mm1_body(x_ref, w_ref, dinv_ref, h_ref):
    h_ref[...] = dinv_ref[...] * jnp.dot(x_ref[...], w_ref[...],
                                         preferred_element_type=jnp.float32)


_mm1 = pl.pallas_call(
    _mm1_body,
    out_shape=jax.ShapeDtypeStruct((N, D), jnp.float32))


def _mid_body(aggp_ref, h_ref, dinv_ref, b_ref, w_ref, h2_ref):
    z = dinv_ref[...] * (aggp_ref[0, :N] + aggp_ref[1, :N] + h_ref[...])
    z = jnp.maximum(z + b_ref[...], 0.0)
    h2_ref[...] = dinv_ref[...] * jnp.dot(z, w_ref[...],
                                          preferred_element_type=jnp.float32)


_mid = pl.pallas_call(
    _mid_body,
    out_shape=jax.ShapeDtypeStruct((N, D), jnp.float32))


def _out_body(aggp_ref, h_ref, dinv_ref, b_ref, x_ref, out_ref):
    z = dinv_ref[...] * (aggp_ref[0, :N] + aggp_ref[1, :N] + h_ref[...])
    out_ref[:, :D] = jnp.maximum(z + b_ref[...], 0.0)
    out_ref[:, D:] = x_ref[...]


_out = pl.pallas_call(
    _out_body,
    out_shape=jax.ShapeDtypeStruct((N, 2 * D), jnp.float32))


def kernel(x, edge_index, edge_weight, W1, b1, W2, b2):
    src = edge_index[0].astype(jnp.int32)
    dst = edge_index[1].astype(jnp.int32)
    w = edge_weight.astype(jnp.float32)
    pad = EPAD - E
    src_p = jnp.pad(src, (0, pad)).reshape(NW * NB, 1, BLK)
    dst_fill = N + (jnp.arange(pad, dtype=jnp.int32) % (NP - N))
    dst_p = jnp.concatenate([dst, dst_fill]).reshape(NW * NB, 1, BLK)
    wbits = lax.bitcast_convert_type(jnp.pad(w, (0, pad)),
                                     jnp.int32).reshape(NW * NB, 1, BLK)
    packed = jnp.concatenate([src_p, dst_p, wbits],
                             axis=1).reshape(NW * NB * 3, BLK)

    degp = _deg_kernel(packed)
    dinv2d = _dinv(degp.reshape(NW, NP // 128, 128))
    dinv_col = dinv2d.reshape(NP, 1)[:N]

    def to_bf16_perm(h):
        hb = (h.astype(jnp.bfloat16)
              .reshape(N, D // 32, 2, 16)
              .transpose(0, 1, 3, 2)
              .reshape(N, D // 2, 2))
        return lax.bitcast_convert_type(hb, jnp.int32)

    h1 = _mm1(x, W1, dinv_col)
    p = _agg_kernel(packed, to_bf16_perm(h1))
    h2 = _mid(p, h1, dinv_col, b1.reshape(1, D), W2)
    q = _agg_kernel(packed, to_bf16_perm(h2))
    return _out(q, h2, dinv_col, b2.reshape(1, D), x)
